# baseline (speedup 1.0000x reference)
import jax
import jax.numpy as jnp
from jax import lax
from jax.experimental import pallas as pl
from jax.experimental.pallas import tpu as pltpu, tpu_sc as plsc

NC = 2
NS = 16
NW = NC * NS

BT = 16384
S = 20
D = 64
N_TOK = BT * S

DP = 128
CHUNK = 128
PACK = CHUNK // 2
N_CHUNKS = N_TOK // CHUNK
CH_PER_W = N_CHUNKS // NW
NBUF = 4
NGRP = CH_PER_W // NBUF


def _emb_body(table_hbm, idx_hbm, off_hbm, out_hbm,
              idx_v, off_v, gbuf, pbuf, gsems, wsems):
    wid = lax.axis_index("s") * NC + lax.axis_index("c")
    c0 = wid * CH_PER_W

    pltpu.sync_copy(idx_hbm.at[pl.ds(c0, CH_PER_W)], idx_v)
    pltpu.sync_copy(off_hbm.at[pl.ds(c0, CH_PER_W)], off_v)

    def gather(j, b):
        return pltpu.make_async_copy(
            table_hbm.at[idx_v.at[j]], gbuf.at[b], gsems.at[b])

    def wout(j, b):
        return pltpu.make_async_copy(
            pbuf.at[b],
            out_hbm.at[pl.ds((c0 + j) * PACK, PACK)],
            wsems.at[b])

    def compact(j, b):
        @pl.loop(0, CHUNK // 16)
        def _(q):
            offv = off_v[j, pl.ds(16 * q, 16)]
            for t in range(16):
                sr = 16 * q + t
                r = 8 * q + t // 2
                h = t & 1
                off = offv[t]
                for s in range(4):
                    pbuf[b, r, pl.ds(64 * h + 16 * s, 16)] = (
                        gbuf[b, sr, pl.ds(off + 16 * s, 16)])

    for b in range(NBUF):
        gather(b, b).start()

    @pl.loop(0, NGRP - 1)
    def _(g):
        j0 = g * NBUF
        for b in range(NBUF):
            j = j0 + b
            gather(j, b).wait()
            compact(j, b)
            gather(j + NBUF, b).start()
            wout(j, b).start()
            wout(j, b).wait()

    for b in range(NBUF):
        j = CH_PER_W - NBUF + b
        gather(j, b).wait()
        compact(j, b)
        wout(j, b).start()
    for b in range(NBUF):
        wout(CH_PER_W - NBUF + b, b).wait()


R_TAB = 1_000_000
FMT_G = 40
FMT_INNER = 500
FMT_ROWS = FMT_G * FMT_INNER


def _fmt_body(src_ref, dst_ref):
    x = src_ref[...].reshape(D, FMT_ROWS)
    row = lax.broadcasted_iota(jnp.int32, (D, D), 0)
    col = lax.broadcasted_iota(jnp.int32, (D, D), 1)
    eye = (row == col).astype(jnp.float32)
    z = lax.dot_general(x, eye, (((0,), (0,)), ((), ())),
                        preferred_element_type=jnp.float32)
    dst_ref[...] = z


_format_table = pl.pallas_call(
    _fmt_body,
    grid=(R_TAB // FMT_ROWS,),
    in_specs=[pl.BlockSpec((D, FMT_G, FMT_INNER), lambda i: (0, i, 0))],
    out_specs=pl.BlockSpec((FMT_ROWS, D), lambda i: (i, 0)),
    out_shape=jax.ShapeDtypeStruct((R_TAB, D), jnp.float32),
)


@jax.jit
def _emb_lookup(idx2d, off2d, table_pairs):
    mesh = plsc.VectorSubcoreMesh(core_axis_name="c", subcore_axis_name="s")
    run = pl.kernel(
        _emb_body,
        out_type=jax.ShapeDtypeStruct((N_TOK // 2, DP), jnp.float32),
        mesh=mesh,
        scratch_types=[
            pltpu.VMEM((CH_PER_W, CHUNK), jnp.int32),
            pltpu.VMEM((CH_PER_W, CHUNK), jnp.int32),
            pltpu.VMEM((NBUF, CHUNK, DP), jnp.float32),
            pltpu.VMEM((NBUF, PACK, DP), jnp.float32),
            pltpu.SemaphoreType.DMA((NBUF,)),
            pltpu.SemaphoreType.DMA((NBUF,)),
        ],
    )
    return run(table_pairs, idx2d, off2d)


def kernel(tokens, emb_weight):
    tok2d = tokens.reshape(N_CHUNKS, CHUNK).astype(jnp.int32)
    idx2d = tok2d >> 1
    off2d = (tok2d & 1) << 6
    emb32 = emb_weight.astype(jnp.float32)
    table = _format_table(emb32.T.reshape(D, R_TAB // FMT_INNER, FMT_INNER))
    table_pairs = table.reshape(R_TAB // 2, DP)
    out = _emb_lookup(idx2d, off2d, table_pairs)
    return out.reshape(BT, S, D)
